# baseline (speedup 1.0000x reference)
import functools

import jax
import jax.numpy as jnp
from jax import lax
from jax.experimental import pallas as pl
from jax.experimental.pallas import tpu as pltpu
from jax.experimental.pallas import tpu_sc as plsc

N_BITS = 32
N_GROUPS = 4
BPG = 8
BATCH = 16384
NC = 2
NS = 16
NW = NC * NS
RPW = BATCH // NW
STEPS = RPW // 16
NPLANES = 8
MAGIC = 0x01020408
TOFF = (0, 256, 768, 1792)
TCOLS = 3840


def _sc_body(bits_hbm, tbl_hbm, out_hbm, bits_v, tbl_v, out_v, sem):
    wid = lax.axis_index("s") * NC + lax.axis_index("c")
    base = pl.multiple_of(wid * RPW, 8)
    c1 = pltpu.async_copy(bits_hbm.at[pl.ds(base, RPW)], bits_v, sem)
    c2 = pltpu.async_copy(tbl_hbm, tbl_v, sem)
    c1.wait()
    c2.wait()
    lanes = lax.iota(jnp.int32, 16)

    @plsc.parallel_loop(0, STEPS, unroll=1)
    def step(i):
        row = i * 16 + lanes
        addrs = []
        for g in range(N_GROUPS):
            w_lo = plsc.load_gather(
                bits_v, [row, jnp.full((16,), 2 * g, jnp.int32)])
            w_hi = plsc.load_gather(
                bits_v, [row, jnp.full((16,), 2 * g + 1, jnp.int32)])
            addrs.append(((w_lo * MAGIC) >> 24)
                         + 16 * ((w_hi * MAGIC) >> 24))
        w0 = plsc.load_gather(tbl_v, [addrs[0]])
        c0 = (w0 >> 8) & 1
        w1 = plsc.load_gather(tbl_v, [addrs[1] + 256 * c0 + TOFF[1]])
        c1b = (w1 >> 8) & 1
        w2 = plsc.load_gather(
            tbl_v, [addrs[2] + 256 * (c0 + 2 * c1b) + TOFF[2]])
        c2b = (w2 >> 8) & 1
        w3 = plsc.load_gather(
            tbl_v, [addrs[3] + 256 * (c0 + 2 * c1b + 4 * c2b) + TOFF[3]])
        out_v[pl.ds(i * 16, 16)] = ((w0 & 255) | ((w1 & 255) << 8)
                                    | ((w2 & 255) << 16) | ((w3 & 255) << 24))

    pltpu.sync_copy(out_v, out_hbm.at[pl.ds(base, RPW)])


@jax.jit
def _mapper(pbits, ptbl):
    mesh = plsc.VectorSubcoreMesh(core_axis_name="c", subcore_axis_name="s")
    f = pl.kernel(
        _sc_body,
        mesh=mesh,
        compiler_params=pltpu.CompilerParams(
            needs_layout_passes=False, use_tc_tiling_on_sc=False),
        out_type=jax.ShapeDtypeStruct((BATCH,), jnp.int32),
        scratch_types=[
            pltpu.VMEM((RPW, NPLANES), jnp.int32),
            pltpu.VMEM((TCOLS,), jnp.int32),
            pltpu.VMEM((RPW,), jnp.int32),
            pltpu.SemaphoreType.DMA,
        ],
    )
    return f(pbits, ptbl)


def kernel(bits, group_mem_0, group_mem_1, group_mem_2, group_mem_3,
           carry_mem_0, carry_mem_1, carry_mem_2):
    pbits = jax.lax.bitcast_convert_type(
        bits.astype(jnp.int8).reshape(BATCH, NPLANES, 4), jnp.int32)
    tblf = jnp.concatenate(
        [group_mem_0, group_mem_1, group_mem_2, group_mem_3], axis=1)
    shifts = jnp.arange(BPG, dtype=jnp.int32)[:, None]
    ptbl = jnp.sum(tblf.astype(jnp.int32) << shifts, axis=0, dtype=jnp.int32)
    carry_col = jnp.concatenate([
        carry_mem_0[0],
        jnp.tile(carry_mem_1[0], 2),
        jnp.tile(carry_mem_2[0], 4),
        jnp.zeros((2048,), jnp.float32),
    ])
    ptbl = ptbl | (carry_col.astype(jnp.int32) << 8)
    outw = _mapper(pbits, ptbl)
    cols = jnp.arange(N_BITS, dtype=jnp.int32)[None, :]
    return ((outw[:, None] >> cols) & 1).astype(jnp.float32)


# speedup vs baseline: 1.4138x; 1.4138x over previous
import functools

import jax
import jax.numpy as jnp
from jax import lax
from jax.experimental import pallas as pl
from jax.experimental.pallas import tpu as pltpu
from jax.experimental.pallas import tpu_sc as plsc

N_BITS = 32
N_GROUPS = 4
BPG = 8
BATCH = 16384
NC = 2
NS = 16
NW = NC * NS
RPW = BATCH // NW
STEPS = RPW // 16
NPLANES = 8
MAGIC = 0x01020408
TOFF = (0, 256, 768, 1792)
TCOLS = 3840


def _sc_body(bits_hbm, tbl_hbm, out_hbm, bits_v, tbl_v, out_v, sem):
    wid = lax.axis_index("s") * NC + lax.axis_index("c")
    base = pl.multiple_of(wid * RPW, 8)
    c1 = pltpu.async_copy(bits_hbm.at[:, pl.ds(base, RPW)], bits_v, sem)
    c2 = pltpu.async_copy(tbl_hbm, tbl_v, sem)
    c1.wait()
    c2.wait()

    @plsc.parallel_loop(0, STEPS, unroll=1)
    def step(i):
        r = i * 16
        addrs = []
        for g in range(N_GROUPS):
            w_lo = bits_v[2 * g, pl.ds(r, 16)]
            w_hi = bits_v[2 * g + 1, pl.ds(r, 16)]
            addrs.append(((w_lo * MAGIC) >> 24)
                         + 16 * ((w_hi * MAGIC) >> 24))
        w0 = plsc.load_gather(tbl_v, [addrs[0]])
        c0 = (w0 >> 8) & 1
        w1 = plsc.load_gather(tbl_v, [addrs[1] + 256 * c0 + TOFF[1]])
        c1b = (w1 >> 8) & 1
        w2 = plsc.load_gather(
            tbl_v, [addrs[2] + 256 * (c0 + 2 * c1b) + TOFF[2]])
        c2b = (w2 >> 8) & 1
        w3 = plsc.load_gather(
            tbl_v, [addrs[3] + 256 * (c0 + 2 * c1b + 4 * c2b) + TOFF[3]])
        out_v[pl.ds(i * 16, 16)] = ((w0 & 255) | ((w1 & 255) << 8)
                                    | ((w2 & 255) << 16) | ((w3 & 255) << 24))

    pltpu.sync_copy(out_v, out_hbm.at[pl.ds(base, RPW)])


@jax.jit
def _mapper(pbits, ptbl):
    mesh = plsc.VectorSubcoreMesh(core_axis_name="c", subcore_axis_name="s")
    f = pl.kernel(
        _sc_body,
        mesh=mesh,
        compiler_params=pltpu.CompilerParams(
            needs_layout_passes=False, use_tc_tiling_on_sc=False),
        out_type=jax.ShapeDtypeStruct((BATCH,), jnp.int32),
        scratch_types=[
            pltpu.VMEM((NPLANES, RPW), jnp.int32),
            pltpu.VMEM((TCOLS,), jnp.int32),
            pltpu.VMEM((RPW,), jnp.int32),
            pltpu.SemaphoreType.DMA,
        ],
    )
    return f(pbits, ptbl)


def kernel(bits, group_mem_0, group_mem_1, group_mem_2, group_mem_3,
           carry_mem_0, carry_mem_1, carry_mem_2):
    pbits = jax.lax.bitcast_convert_type(
        bits.astype(jnp.int8).reshape(BATCH, NPLANES, 4), jnp.int32).T
    tblf = jnp.concatenate(
        [group_mem_0, group_mem_1, group_mem_2, group_mem_3], axis=1)
    shifts = jnp.arange(BPG, dtype=jnp.int32)[:, None]
    ptbl = jnp.sum(tblf.astype(jnp.int32) << shifts, axis=0, dtype=jnp.int32)
    carry_col = jnp.concatenate([
        carry_mem_0[0],
        jnp.tile(carry_mem_1[0], 2),
        jnp.tile(carry_mem_2[0], 4),
        jnp.zeros((2048,), jnp.float32),
    ])
    ptbl = ptbl | (carry_col.astype(jnp.int32) << 8)
    outw = _mapper(pbits, ptbl)
    cols = jnp.arange(N_BITS, dtype=jnp.int32)[None, :]
    return ((outw[:, None] >> cols) & 1).astype(jnp.float32)


# speedup vs baseline: 1.5258x; 1.0792x over previous
import functools

import jax
import jax.numpy as jnp
from jax import lax
from jax.experimental import pallas as pl
from jax.experimental.pallas import tpu as pltpu
from jax.experimental.pallas import tpu_sc as plsc

N_BITS = 32
N_GROUPS = 4
BPG = 8
BATCH = 16384
NC = 1
NS = 16
NW = NC * NS
RPW = BATCH // NW
STEPS = RPW // 16
NPLANES = 8
MAGIC = 0x01020408
TOFF = (0, 256, 768, 1792)
TCOLS = 3840


def _sc_body(bits_hbm, tbl_hbm, out_hbm, bits_v, tbl_v, out_v, sem):
    wid = lax.axis_index("s") * NC + lax.axis_index("c")
    base = pl.multiple_of(wid * RPW, 8)
    c1 = pltpu.async_copy(bits_hbm.at[:, pl.ds(base, RPW)], bits_v, sem)
    c2 = pltpu.async_copy(tbl_hbm, tbl_v, sem)
    c1.wait()
    c2.wait()

    @plsc.parallel_loop(0, STEPS, unroll=1)
    def step(i):
        r = i * 16
        addrs = []
        for g in range(N_GROUPS):
            w_lo = bits_v[2 * g, pl.ds(r, 16)]
            w_hi = bits_v[2 * g + 1, pl.ds(r, 16)]
            addrs.append(((w_lo * MAGIC) >> 24)
                         + 16 * ((w_hi * MAGIC) >> 24))
        w0 = plsc.load_gather(tbl_v, [addrs[0]])
        c0 = (w0 >> 8) & 1
        w1 = plsc.load_gather(tbl_v, [addrs[1] + 256 * c0 + TOFF[1]])
        c1b = (w1 >> 8) & 1
        w2 = plsc.load_gather(
            tbl_v, [addrs[2] + 256 * (c0 + 2 * c1b) + TOFF[2]])
        c2b = (w2 >> 8) & 1
        w3 = plsc.load_gather(
            tbl_v, [addrs[3] + 256 * (c0 + 2 * c1b + 4 * c2b) + TOFF[3]])
        out_v[pl.ds(i * 16, 16)] = ((w0 & 255) | ((w1 & 255) << 8)
                                    | ((w2 & 255) << 16) | ((w3 & 255) << 24))

    pltpu.sync_copy(out_v, out_hbm.at[pl.ds(base, RPW)])


@jax.jit
def _mapper(pbits, ptbl):
    mesh = plsc.VectorSubcoreMesh(core_axis_name="c", subcore_axis_name="s", num_cores=1)
    f = pl.kernel(
        _sc_body,
        mesh=mesh,
        compiler_params=pltpu.CompilerParams(
            needs_layout_passes=False, use_tc_tiling_on_sc=False),
        out_type=jax.ShapeDtypeStruct((BATCH,), jnp.int32),
        scratch_types=[
            pltpu.VMEM((NPLANES, RPW), jnp.int32),
            pltpu.VMEM((TCOLS,), jnp.int32),
            pltpu.VMEM((RPW,), jnp.int32),
            pltpu.SemaphoreType.DMA,
        ],
    )
    return f(pbits, ptbl)


def kernel(bits, group_mem_0, group_mem_1, group_mem_2, group_mem_3,
           carry_mem_0, carry_mem_1, carry_mem_2):
    pbits = jax.lax.bitcast_convert_type(
        bits.astype(jnp.int8).reshape(BATCH, NPLANES, 4), jnp.int32).T
    tblf = jnp.concatenate(
        [group_mem_0, group_mem_1, group_mem_2, group_mem_3], axis=1)
    shifts = jnp.arange(BPG, dtype=jnp.int32)[:, None]
    ptbl = jnp.sum(tblf.astype(jnp.int32) << shifts, axis=0, dtype=jnp.int32)
    carry_col = jnp.concatenate([
        carry_mem_0[0],
        jnp.tile(carry_mem_1[0], 2),
        jnp.tile(carry_mem_2[0], 4),
        jnp.zeros((2048,), jnp.float32),
    ])
    ptbl = ptbl | (carry_col.astype(jnp.int32) << 8)
    outw = _mapper(pbits, ptbl)
    cols = jnp.arange(N_BITS, dtype=jnp.int32)[None, :]
    return ((outw[:, None] >> cols) & 1).astype(jnp.float32)
